# baseline (speedup 1.0000x reference)
import functools

import jax
import jax.numpy as jnp
from jax import lax
from jax.experimental import pallas as pl
from jax.experimental.pallas import tpu as pltpu
from jax.experimental.pallas import tpu_sc as plsc

BMG = 512
BAL = 0.01


def _router_kernel(feat_ref, wr_ref, br_ref,
                   eids_ref, prel_ref, wts_ref, offs_ref, bexp_ref, loss_ref,
                   ctr_ref, psum_ref, lt_ref,
                   *, n_tokens, n_exp, nbg):
    i = pl.program_id(0)
    nsteps = pl.num_programs(0)
    bm = feat_ref.shape[0]

    feat = feat_ref[...].astype(jnp.bfloat16)
    logits = jnp.dot(feat, wr_ref[...].astype(jnp.bfloat16),
                     preferred_element_type=jnp.float32)
    logits = logits + br_ref[...]
    m = jnp.max(logits, axis=1, keepdims=True)
    ex = jnp.exp(logits - m)
    prob = ex / jnp.sum(ex, axis=1, keepdims=True)

    eidx = lax.broadcasted_iota(jnp.int32, prob.shape, 1)
    i1 = jnp.argmax(prob, axis=1).astype(jnp.int32)
    v1 = jnp.max(prob, axis=1)
    masked = jnp.where(eidx == i1[:, None], -1.0, prob)
    i2 = jnp.argmax(masked, axis=1).astype(jnp.int32)
    v2 = jnp.max(masked, axis=1)
    denom = jnp.maximum(v1 + v2, 1e-9)
    w1 = v1 / denom
    w2 = v2 / denom

    a1 = (eidx == i1[:, None]).astype(jnp.bfloat16)
    a2 = (eidx == i2[:, None]).astype(jnp.bfloat16)

    @pl.when(i == 0)
    def _():
        ctr_ref[...] = jnp.zeros_like(ctr_ref)
        psum_ref[...] = jnp.zeros_like(psum_ref)
        r = lax.broadcasted_iota(jnp.int32, (2 * bm, 2 * bm), 0)
        c = lax.broadcasted_iota(jnp.int32, (2 * bm, 2 * bm), 1)
        lt_ref[...] = (c < r).astype(jnp.bfloat16)

    h = jnp.concatenate([a1, a2], axis=0)
    s = jnp.dot(lt_ref[...], h, preferred_element_type=jnp.float32)
    ctr = ctr_ref[...]
    a1f = a1.astype(jnp.float32)
    a2f = a2.astype(jnp.float32)
    rank1 = jnp.sum((s[:bm] + ctr) * a1f, axis=1)
    rank2 = jnp.sum((s[bm:] + ctr) * a2f, axis=1)

    eids_ref[...] = jnp.concatenate([i1[:, None], i2[:, None]], axis=1)
    prel_ref[...] = jnp.concatenate(
        [rank1[:, None], rank2[:, None]], axis=1).astype(jnp.int32)
    wts_ref[...] = jnp.concatenate([w1[:, None], w2[:, None]], axis=1)

    ctr_ref[...] = ctr + jnp.sum(h, axis=0, keepdims=True).astype(jnp.float32)
    psum_ref[...] = psum_ref[...] + jnp.sum(prob, axis=0, keepdims=True)

    @pl.when(i == nsteps - 1)
    def _():
        cnt = ctr_ref[...]
        padded = jnp.ceil(cnt / BMG) * BMG
        rr = lax.broadcasted_iota(jnp.int32, (n_exp, n_exp), 0)
        cc = lax.broadcasted_iota(jnp.int32, (n_exp, n_exp), 1)
        ut = (rr < cc).astype(jnp.float32)
        offs = jnp.dot(padded, ut, preferred_element_type=jnp.float32)
        offs_ref[...] = offs.astype(jnp.int32)
        biota = lax.broadcasted_iota(jnp.int32, (1, nbg), 1) * BMG
        bexp = jnp.zeros((1, nbg), jnp.int32)
        for e in range(1, n_exp):
            oe = offs[0, e].astype(jnp.int32)
            bexp = bexp + (biota >= oe).astype(jnp.int32)
        bexp_ref[...] = bexp
        pi = psum_ref[...] / float(n_tokens)
        ent = jnp.sum(pi * jnp.log(jnp.maximum(pi, 1e-9)),
                      axis=1, keepdims=True)
        loss_ref[...] = BAL * (ent + jnp.log(float(n_exp)))


def _run_router(features, Wr, br, nbg):
    n, d = features.shape
    e = Wr.shape[1]
    bm = 512
    grid = (n // bm,)
    return pl.pallas_call(
        functools.partial(_router_kernel, n_tokens=n, n_exp=e, nbg=nbg),
        grid=grid,
        in_specs=[
            pl.BlockSpec((bm, d), lambda i: (i, 0)),
            pl.BlockSpec((d, e), lambda i: (0, 0)),
            pl.BlockSpec((1, e), lambda i: (0, 0)),
        ],
        out_specs=[
            pl.BlockSpec((bm, 2), lambda i: (i, 0)),
            pl.BlockSpec((bm, 2), lambda i: (i, 0)),
            pl.BlockSpec((bm, 2), lambda i: (i, 0)),
            pl.BlockSpec((1, e), lambda i: (0, 0)),
            pl.BlockSpec((1, nbg), lambda i: (0, 0)),
            pl.BlockSpec((1, 1), lambda i: (0, 0)),
        ],
        out_shape=[
            jax.ShapeDtypeStruct((n, 2), jnp.int32),
            jax.ShapeDtypeStruct((n, 2), jnp.int32),
            jax.ShapeDtypeStruct((n, 2), jnp.float32),
            jax.ShapeDtypeStruct((1, e), jnp.int32),
            jax.ShapeDtypeStruct((1, nbg), jnp.int32),
            jax.ShapeDtypeStruct((1, 1), jnp.float32),
        ],
        scratch_shapes=[
            pltpu.VMEM((1, e), jnp.float32),
            pltpu.VMEM((1, e), jnp.float32),
            pltpu.VMEM((2 * bm, 2 * bm), jnp.bfloat16),
        ],
    )(features, Wr, br.reshape(1, e))


def _posfix_kernel(eids_ref, prel_ref, wts_ref, offs_ref, pos_ref, wexp_ref,
                   *, n_exp):
    bm = eids_ref.shape[0]
    iota8 = lax.broadcasted_iota(jnp.int32, (bm, n_exp), 1)
    offs = offs_ref[...]
    pcols = []
    for k in range(2):
        cmp = iota8 == eids_ref[:, k:k + 1]
        offk = jnp.sum(jnp.where(cmp, offs, 0), axis=1, keepdims=True)
        pcols.append(offk + prel_ref[:, k:k + 1])
    pos_ref[...] = jnp.concatenate(pcols, axis=1)
    w0 = jnp.broadcast_to(wts_ref[:, 0:1], (bm, 16))
    w1 = jnp.broadcast_to(wts_ref[:, 1:2], (bm, 16))
    wexp_ref[...] = jnp.concatenate([w0, w1], axis=1)


def _run_posfix(eids, prel, wts, offs, n_exp):
    n = eids.shape[0]
    bm = 512
    return pl.pallas_call(
        functools.partial(_posfix_kernel, n_exp=n_exp),
        grid=(n // bm,),
        in_specs=[
            pl.BlockSpec((bm, 2), lambda i: (i, 0)),
            pl.BlockSpec((bm, 2), lambda i: (i, 0)),
            pl.BlockSpec((bm, 2), lambda i: (i, 0)),
            pl.BlockSpec((1, n_exp), lambda i: (0, 0)),
        ],
        out_specs=[
            pl.BlockSpec((bm, 2), lambda i: (i, 0)),
            pl.BlockSpec((bm, 32), lambda i: (i, 0)),
        ],
        out_shape=[
            jax.ShapeDtypeStruct((n, 2), jnp.int32),
            jax.ShapeDtypeStruct((n, 32), jnp.float32),
        ],
    )(eids, prel, wts, offs)


def _make_dispatch(n, d, r_pad):
    na = 2 * n
    nw = 32
    s_per_w = na // nw
    nchunks = s_per_w // 128
    mesh = plsc.VectorSubcoreMesh(core_axis_name="c", subcore_axis_name="s")

    @functools.partial(
        pl.kernel, mesh=mesh,
        out_type=jax.ShapeDtypeStruct((r_pad, d), jnp.float32),
        scratch_types=[
            pltpu.VMEM((nchunks, 128), jnp.int32),
            pltpu.VMEM((nchunks, 128), jnp.int32),
            pltpu.VMEM((128, d), jnp.float32),
            pltpu.SemaphoreType.DMA,
        ],
    )
    def dispatch(pos_hbm, feat_hbm, a_hbm, pos_v, tok_v, rows_v, sem):
        wid = lax.axis_index("s") * 2 + lax.axis_index("c")
        abase = wid * s_per_w
        for c in range(nchunks):
            pltpu.sync_copy(pos_hbm.at[pl.ds(abase + c * 128, 128)],
                            pos_v.at[c])
            for j in range(8):
                b = c * 128 + j * 16
                tok = lax.shift_right_logical(
                    abase + b + lax.iota(jnp.int32, 16), 1)
                tok_v[c, pl.ds(j * 16, 16)] = tok
            pltpu.async_copy(feat_hbm.at[tok_v.at[c]], rows_v, sem).wait()
            pltpu.async_copy(rows_v, a_hbm.at[pos_v.at[c]], sem).wait()

    return dispatch


def _gmm_kernel(bexp_ref, a_ref, we_ref, be_ref, buf_ref, webf_ref):
    i = pl.program_id(0)
    g = bexp_ref[i]

    @pl.when(i == 0)
    def _():
        webf_ref[...] = we_ref[...].astype(jnp.bfloat16)

    a = a_ref[...].astype(jnp.bfloat16)
    acc = jnp.dot(a, webf_ref[g], preferred_element_type=jnp.float32)
    buf_ref[...] = acc + be_ref[g]


def _run_gmm(bexp, a, We, be, r_pad, nbg):
    e, d, o = We.shape
    grid_spec = pltpu.PrefetchScalarGridSpec(
        num_scalar_prefetch=1,
        grid=(nbg,),
        in_specs=[
            pl.BlockSpec((BMG, d), lambda i, b: (i, 0)),
            pl.BlockSpec((e, d, o), lambda i, b: (0, 0, 0)),
            pl.BlockSpec((e, o), lambda i, b: (0, 0)),
        ],
        out_specs=pl.BlockSpec((BMG, o), lambda i, b: (i, 0)),
        scratch_shapes=[pltpu.VMEM((e, d, o), jnp.bfloat16)],
    )
    return pl.pallas_call(
        _gmm_kernel,
        grid_spec=grid_spec,
        out_shape=jax.ShapeDtypeStruct((r_pad, o), jnp.float32),
    )(bexp, a, We, be)


def _make_combine(n, o, r_pad):
    nw = 32
    t_per_w = n // nw
    tchunk = 32
    nchunks = t_per_w // tchunk
    mesh = plsc.VectorSubcoreMesh(core_axis_name="c", subcore_axis_name="s")

    @functools.partial(
        pl.kernel, mesh=mesh,
        out_type=jax.ShapeDtypeStruct((n, o), jnp.float32),
        scratch_types=[
            pltpu.VMEM((32 * t_per_w,), jnp.float32),
            pltpu.VMEM((nchunks, 2 * tchunk), jnp.int32),
            pltpu.VMEM((2 * tchunk, o), jnp.float32),
            pltpu.VMEM((tchunk, o), jnp.float32),
            pltpu.SemaphoreType.DMA,
        ],
    )
    def combine(buf_hbm, pos_hbm, wexp_hbm, out_hbm,
                wt_v, pidx_v, rows_v, out_v, sem):
        wid = lax.axis_index("s") * 2 + lax.axis_index("c")
        tbase = wid * t_per_w
        pltpu.sync_copy(wexp_hbm.at[pl.ds(32 * tbase, 32 * t_per_w)], wt_v)
        for c in range(nchunks):
            pltpu.sync_copy(
                pos_hbm.at[pl.ds(2 * tbase + c * 2 * tchunk, 2 * tchunk)],
                pidx_v.at[c])
            pltpu.async_copy(buf_hbm.at[pidx_v.at[c]], rows_v, sem).wait()

            def body(t, _):
                wb = 32 * (c * tchunk + t)
                w0 = wt_v[pl.ds(wb, 16)]
                w1 = wt_v[pl.ds(wb + 16, 16)]

                def inner(sl, _):
                    a = rows_v[2 * t, pl.ds(sl * 16, 16)]
                    b = rows_v[2 * t + 1, pl.ds(sl * 16, 16)]
                    out_v[t, pl.ds(sl * 16, 16)] = a * w0 + b * w1
                    return 0
                return lax.fori_loop(0, o // 16, inner, 0, unroll=4)
            lax.fori_loop(0, tchunk, body, 0)
            pltpu.sync_copy(out_v,
                            out_hbm.at[pl.ds(tbase + c * tchunk, tchunk)])

    return combine


def kernel(features, Wr, br, We, be):
    n, d = features.shape
    e, _, o = We.shape
    r_pad = 2 * n + e * BMG
    nbg = r_pad // BMG

    eids, prel, wts, offs, bexp, loss = _run_router(features, Wr, br, nbg)
    pos, wexp = _run_posfix(eids, prel, wts, offs, e)
    a = _make_dispatch(n, d, r_pad)(pos.reshape(2 * n), features)
    buf = _run_gmm(bexp.reshape(nbg), a, We, be, r_pad, nbg)
    logits = _make_combine(n, o, r_pad)(
        buf, pos.reshape(2 * n), wexp.reshape(32 * n))
    return logits, loss.reshape(())


# speedup vs baseline: 3.0228x; 3.0228x over previous
import functools

import jax
import jax.numpy as jnp
from jax.experimental import pallas as pl
from jax.experimental.pallas import tpu as pltpu


def _moe_block_kernel(feat_ref, wr_ref, br_ref, we_ref, be_ref,
                      out_ref, psum_ref, loss_ref, webf_ref,
                      *, n_tokens, n_exp, bal):
    i = pl.program_id(0)
    nsteps = pl.num_programs(0)

    @pl.when(i == 0)
    def _():
        webf_ref[...] = we_ref[...].astype(jnp.bfloat16)

    feat = feat_ref[...].astype(jnp.bfloat16)
    logits = jnp.dot(feat, wr_ref[...].astype(jnp.bfloat16),
                     preferred_element_type=jnp.float32)
    logits = logits + br_ref[...]
    m = jnp.max(logits, axis=1, keepdims=True)
    ex = jnp.exp(logits - m)
    prob = ex / jnp.sum(ex, axis=1, keepdims=True)

    eidx = jax.lax.broadcasted_iota(jnp.int32, prob.shape, 1)
    i1 = jnp.argmax(prob, axis=1).astype(jnp.int32)
    v1 = jnp.max(prob, axis=1)
    masked = jnp.where(eidx == i1[:, None], -1.0, prob)
    i2 = jnp.argmax(masked, axis=1).astype(jnp.int32)
    v2 = jnp.max(masked, axis=1)
    denom = jnp.maximum(v1 + v2, 1e-9)
    w1 = v1 / denom
    w2 = v2 / denom
    w8 = (jnp.where(eidx == i1[:, None], w1[:, None], 0.0)
          + jnp.where(eidx == i2[:, None], w2[:, None], 0.0))

    acc = jnp.dot(w8, be_ref[...], preferred_element_type=jnp.float32)
    for e in range(n_exp):
        neo = jnp.dot(feat, webf_ref[e], preferred_element_type=jnp.float32)
        acc = acc + w8[:, e:e + 1] * neo
    out_ref[...] = acc

    block_psum = jnp.sum(prob, axis=0, keepdims=True)
    @pl.when(i == 0)
    def _():
        psum_ref[...] = block_psum
        loss_ref[...] = jnp.zeros_like(loss_ref)

    @pl.when(i > 0)
    def _():
        psum_ref[...] = psum_ref[...] + block_psum

    @pl.when(i == nsteps - 1)
    def _():
        pi = psum_ref[...] / float(n_tokens)
        ent = jnp.sum(pi * jnp.log(jnp.maximum(pi, 1e-9)),
                      axis=1, keepdims=True)
        loss_ref[...] = bal * (ent + jnp.log(float(n_exp)))


def kernel(features, Wr, br, We, be):
    n, d = features.shape
    e, _, o = We.shape
    bm = min(1024, n)
    grid = (n // bm,)

    out, _, loss = pl.pallas_call(
        functools.partial(_moe_block_kernel, n_tokens=n, n_exp=e, bal=0.01),
        grid=grid,
        in_specs=[
            pl.BlockSpec((bm, d), lambda i: (i, 0)),
            pl.BlockSpec((d, e), lambda i: (0, 0)),
            pl.BlockSpec((1, e), lambda i: (0, 0)),
            pl.BlockSpec((e, d, o), lambda i: (0, 0, 0)),
            pl.BlockSpec((e, o), lambda i: (0, 0)),
        ],
        out_specs=[
            pl.BlockSpec((bm, o), lambda i: (i, 0)),
            pl.BlockSpec((1, e), lambda i: (0, 0)),
            pl.BlockSpec((1, 1), lambda i: (0, 0)),
        ],
        out_shape=[
            jax.ShapeDtypeStruct((n, o), jnp.float32),
            jax.ShapeDtypeStruct((1, e), jnp.float32),
            jax.ShapeDtypeStruct((1, 1), jnp.float32),
        ],
        scratch_shapes=[pltpu.VMEM((e, d, o), jnp.bfloat16)],
    )(features, Wr, br.reshape(1, e), We, be)
    return out, loss.reshape(())
